# Initial kernel scaffold; baseline (speedup 1.0000x reference)
import jax
import jax.numpy as jnp
from jax.experimental import pallas as pl


def kernel(input, pe, scale_param):
    raise NotImplementedError("write your pallas kernel here")


# speedup vs baseline: 1.7371x; 1.7371x over previous
import jax
import jax.numpy as jnp
from jax.experimental import pallas as pl


S_BLK = 256


def _pe_add_kernel(scale_ref, in_ref, pe_ref, out_ref):
    s = scale_ref[0]
    out_ref[...] = in_ref[...] + s * pe_ref[...][None, :, :]


def kernel(input, pe, scale_param):
    batch, seq, dim = input.shape
    grid = (seq // S_BLK,)
    return pl.pallas_call(
        _pe_add_kernel,
        grid=grid,
        in_specs=[
            pl.BlockSpec((1,), lambda i: (0,)),
            pl.BlockSpec((batch, S_BLK, dim), lambda i: (0, i, 0)),
            pl.BlockSpec((S_BLK, dim), lambda i: (i, 0)),
        ],
        out_specs=pl.BlockSpec((batch, S_BLK, dim), lambda i: (0, i, 0)),
        out_shape=jax.ShapeDtypeStruct((batch, seq, dim), input.dtype),
    )(scale_param, input, pe[:seq])


# speedup vs baseline: 1.7864x; 1.0284x over previous
import jax
import jax.numpy as jnp
from jax.experimental import pallas as pl


S_BLK = 512


def _pe_add_kernel(scale_ref, in_ref, pe_ref, out_ref):
    s = scale_ref[0]
    out_ref[...] = in_ref[...] + s * pe_ref[...][None, :, :]


def kernel(input, pe, scale_param):
    batch, seq, dim = input.shape
    grid = (seq // S_BLK,)
    return pl.pallas_call(
        _pe_add_kernel,
        grid=grid,
        in_specs=[
            pl.BlockSpec((1,), lambda i: (0,)),
            pl.BlockSpec((batch, S_BLK, dim), lambda i: (0, i, 0)),
            pl.BlockSpec((S_BLK, dim), lambda i: (i, 0)),
        ],
        out_specs=pl.BlockSpec((batch, S_BLK, dim), lambda i: (0, i, 0)),
        out_shape=jax.ShapeDtypeStruct((batch, seq, dim), input.dtype),
    )(scale_param, input, pe[:seq])
